# baseline (speedup 1.0000x reference)
import jax
import jax.numpy as jnp
import numpy as np
from jax.experimental import pallas as pl
from jax.experimental.pallas import tpu as pltpu
from jax.experimental.pallas import tpu_sc as plsc

B = 4096
VOCAB = 1000000
D = 128
SLOTS = 26
SLOTP = 32
NF = SLOTS + 1
DENSE = 13
_PAIR = np.array(
    [i * NF + j for j in range(1, NF) for i in range(j)], dtype=np.int32
)

_GATHER_WIN = 128

HALVES = 2
BH = B // HALVES
CH = 256
NCH = BH // CH
SUB = 16
SUBR = CH // SUB
NBUF = 5
PREF = 4


def _sc_gather(table, idx_flat):
    n = idx_flat.shape[1]
    mesh = plsc.VectorSubcoreMesh(core_axis_name="core", subcore_axis_name="subcore")

    @pl.kernel(out_type=jax.ShapeDtypeStruct((n, D), table.dtype), mesh=mesh)
    def gather_kernel(tab_hbm, i_hbm, o_hbm):
        def body(i_vmem, o_vmem):
            pltpu.sync_copy(tab_hbm.at[i_vmem.at[0]], o_vmem)

        pltpu.emit_pipeline(
            body,
            grid=(n // _GATHER_WIN,),
            in_specs=[pl.BlockSpec((1, _GATHER_WIN), index_map=lambda i: (0, i))],
            out_specs=[pl.BlockSpec((_GATHER_WIN, D), index_map=lambda i: (i, 0))],
            core_axis_name=("core", "subcore"),
            dimension_semantics=(pltpu.PARALLEL,),
        )(i_hbm, o_hbm)

    return gather_kernel(table, idx_flat)


def _dense_body(num_ref, emb_hbm, bw0, bb0, bw1, bb1, bw2, bb2,
                w0x, w0e, b0, w1, b1, w2, b2, w3, b3, w4, b4,
                out_ref, ebuf, sems):
    f32 = jnp.float32
    bf16 = jnp.bfloat16

    def copy(c, s):
        bslot = c % NBUF
        return pltpu.make_async_copy(
            emb_hbm.at[pl.ds(c * CH + s * SUBR, SUBR)],
            ebuf.at[bslot, pl.ds(s * SUBR, SUBR)],
            sems.at[bslot, s],
        )

    for c in range(PREF):
        for s in range(SUB):
            copy(c, s).start()

    for c in range(NCH):
        if c + PREF < NCH:
            for s in range(SUB):
                copy(c + PREF, s).start()
        for s in range(SUB):
            copy(c, s).wait()

        x = num_ref[pl.ds(c * CH, CH), :]
        x = jax.nn.relu(jnp.dot(x, bw0[...], preferred_element_type=f32) + bb0[...])
        x = jax.nn.relu(jnp.dot(x.astype(bf16), bw1[...], preferred_element_type=f32) + bb1[...])
        x = jax.nn.relu(jnp.dot(x.astype(bf16), bw2[...], preferred_element_type=f32) + bb2[...])

        emb = ebuf[c % NBUF, :, :NF, :]
        sl = jax.lax.broadcasted_iota(jnp.int32, (CH, NF, D), 1)
        cat = jnp.where(sl == SLOTS, x[:, None, :], emb)
        catb = cat.astype(bf16)
        gram = jax.lax.dot_general(
            catb, catb, (((2,), (2,)), ((0,), (0,))), preferred_element_type=f32
        )
        gf = gram.reshape(CH, NF * NF).astype(bf16)

        h = jnp.dot(x.astype(bf16), w0x[...], preferred_element_type=f32)
        h = h + jnp.dot(gf, w0e[...], preferred_element_type=f32) + b0[...]
        h = jax.nn.relu(h)
        h = jax.nn.relu(jnp.dot(h.astype(bf16), w1[...], preferred_element_type=f32) + b1[...])
        h = jax.nn.relu(jnp.dot(h.astype(bf16), w2[...], preferred_element_type=f32) + b2[...])
        h = jax.nn.relu(jnp.dot(h.astype(bf16), w3[...], preferred_element_type=f32) + b3[...])
        logit = jnp.dot(h.astype(bf16), w4[...], preferred_element_type=f32) + b4[...]
        out_ref[pl.ds(c * CH, CH), :] = jax.nn.sigmoid(logit)


def _dense_call(interpret=False):
    def _vmem_full(shape):
        nd = len(shape)
        return pl.BlockSpec(shape, (lambda i: (0,) * nd))

    def run(num, emb_flat, bw0, bb0, bw1, bb1, bw2, bb2,
            w0x, w0e, b0, w1, b1, w2, b2, w3, b3, w4, b4):
        weight_args = (bw0, bb0, bw1, bb1, bw2, bb2,
                       w0x, w0e, b0, w1, b1, w2, b2, w3, b3, w4, b4)
        return pl.pallas_call(
            _dense_body,
            grid=(1,),
            in_specs=[
                pl.BlockSpec((BH, DENSE), lambda i: (0, 0)),
                pl.BlockSpec(memory_space=pltpu.MemorySpace.HBM),
            ] + [_vmem_full(w.shape) for w in weight_args],
            out_specs=pl.BlockSpec((BH, 1), lambda i: (0, 0)),
            out_shape=jax.ShapeDtypeStruct((BH, 1), jnp.float32),
            scratch_shapes=[
                pltpu.VMEM((NBUF, CH, SLOTP, D), jnp.float32),
                pltpu.SemaphoreType.DMA((NBUF, SUB)),
            ],
            interpret=interpret,
        )(num, emb_flat, *weight_args)

    return run


def kernel(categorical_features, numerical_features, table,
           bot_W0, bot_b0, bot_W1, bot_b1, bot_W2, bot_b2,
           top_W0, top_b0, top_W1, top_b1, top_W2, top_b2,
           top_W3, top_b3, top_W4, top_b4):
    cf = categorical_features.astype(jnp.int32)
    pad_idx = (
        jnp.arange(B * (SLOTP - SLOTS), dtype=jnp.int32).reshape(B, SLOTP - SLOTS)
        % VOCAB
    )
    idx = jnp.concatenate([cf, pad_idx], axis=1)

    bf16 = jnp.bfloat16
    w0x = top_W0[:D].astype(bf16)
    w0e = (
        jnp.zeros((NF * NF, top_W0.shape[1]), jnp.float32)
        .at[_PAIR].set(top_W0[D:])
        .astype(bf16)
    )

    run = _dense_call()
    weights = (
        bot_W0, bot_b0.reshape(1, -1), bot_W1.astype(bf16), bot_b1.reshape(1, -1),
        bot_W2.astype(bf16), bot_b2.reshape(1, -1),
        w0x, w0e, top_b0.reshape(1, -1),
        top_W1.astype(bf16), top_b1.reshape(1, -1),
        top_W2.astype(bf16), top_b2.reshape(1, -1),
        top_W3.astype(bf16), top_b3.reshape(1, -1),
        top_W4.astype(bf16), top_b4.reshape(1, -1),
    )
    outs = []
    for k in range(HALVES):
        idx_k = idx[k * BH:(k + 1) * BH].reshape(1, BH * SLOTP)
        emb_k = _sc_gather(table, idx_k)
        emb_k3 = emb_k.reshape(BH, SLOTP, D)
        num_k = numerical_features[k * BH:(k + 1) * BH]
        outs.append(run(num_k, emb_k3, *weights))
    return jnp.concatenate(outs, axis=0)


# speedup vs baseline: 1.0009x; 1.0009x over previous
import jax
import jax.numpy as jnp
import numpy as np
from jax.experimental import pallas as pl
from jax.experimental.pallas import tpu as pltpu
from jax.experimental.pallas import tpu_sc as plsc

B = 4096
VOCAB = 1000000
D = 128
SLOTS = 26
SLOTP = 32
NF = SLOTS + 1
DENSE = 13
_PAIR = np.array(
    [i * NF + j for j in range(1, NF) for i in range(j)], dtype=np.int32
)

_GATHER_WIN = 128

HALVES = 2
BH = B // HALVES
CH = 256
NCH = BH // CH
SUB = 16
SUBR = CH // SUB
NBUF = 5
PREF = 4


def _sc_gather(table, idx_flat):
    n = idx_flat.shape[1]
    mesh = plsc.VectorSubcoreMesh(core_axis_name="core", subcore_axis_name="subcore")

    @pl.kernel(out_type=jax.ShapeDtypeStruct((n, D), table.dtype), mesh=mesh)
    def gather_kernel(tab_hbm, i_hbm, o_hbm):
        def body(i_vmem, o_vmem):
            pltpu.sync_copy(tab_hbm.at[i_vmem.at[0]], o_vmem)

        pltpu.emit_pipeline(
            body,
            grid=(n // _GATHER_WIN,),
            in_specs=[pl.BlockSpec((1, _GATHER_WIN), index_map=lambda i: (0, i))],
            out_specs=[pl.BlockSpec((_GATHER_WIN, D), index_map=lambda i: (i, 0))],
            core_axis_name=("core", "subcore"),
            dimension_semantics=(pltpu.PARALLEL,),
        )(i_hbm, o_hbm)

    return gather_kernel(table, idx_flat)


def _dense_body(num_ref, emb_hbm, bw0, bb0, bw1, bb1, bw2, bb2,
                w0x, w0e, b0, w1, b1, w2, b2, w3, b3, w4, b4,
                out_ref, ebuf, sems):
    f32 = jnp.float32
    bf16 = jnp.bfloat16

    def copy(c, s):
        bslot = c % NBUF
        return pltpu.make_async_copy(
            emb_hbm.at[pl.ds(c * CH + s * SUBR, SUBR)],
            ebuf.at[bslot, pl.ds(s * SUBR, SUBR)],
            sems.at[bslot, s],
        )

    for c in range(PREF):
        for s in range(SUB):
            copy(c, s).start()

    for c in range(NCH):
        if c + PREF < NCH:
            for s in range(SUB):
                copy(c + PREF, s).start()
        for s in range(SUB):
            copy(c, s).wait()

        x = num_ref[pl.ds(c * CH, CH), :]
        x = jax.nn.relu(jnp.dot(x, bw0[...], preferred_element_type=f32) + bb0[...])
        x = jax.nn.relu(jnp.dot(x.astype(bf16), bw1[...], preferred_element_type=f32) + bb1[...])
        x = jax.nn.relu(jnp.dot(x.astype(bf16), bw2[...], preferred_element_type=f32) + bb2[...])

        emb = ebuf[c % NBUF, :, :NF, :].astype(bf16)
        sl = jax.lax.broadcasted_iota(jnp.int32, (CH, NF, D), 1)
        catb = jnp.where(sl == SLOTS, x.astype(bf16)[:, None, :], emb)
        gram = jax.lax.dot_general(
            catb, catb, (((2,), (2,)), ((0,), (0,))), preferred_element_type=f32
        )
        gf = gram.astype(bf16).reshape(CH, NF * NF)

        h = jnp.dot(x.astype(bf16), w0x[...], preferred_element_type=f32)
        h = h + jnp.dot(gf, w0e[...], preferred_element_type=f32) + b0[...]
        h = jax.nn.relu(h)
        h = jax.nn.relu(jnp.dot(h.astype(bf16), w1[...], preferred_element_type=f32) + b1[...])
        h = jax.nn.relu(jnp.dot(h.astype(bf16), w2[...], preferred_element_type=f32) + b2[...])
        h = jax.nn.relu(jnp.dot(h.astype(bf16), w3[...], preferred_element_type=f32) + b3[...])
        logit = jnp.dot(h.astype(bf16), w4[...], preferred_element_type=f32) + b4[...]
        out_ref[pl.ds(c * CH, CH), :] = jax.nn.sigmoid(logit)


def _dense_call(interpret=False):
    def _vmem_full(shape):
        nd = len(shape)
        return pl.BlockSpec(shape, (lambda i: (0,) * nd))

    def run(num, emb_flat, bw0, bb0, bw1, bb1, bw2, bb2,
            w0x, w0e, b0, w1, b1, w2, b2, w3, b3, w4, b4):
        weight_args = (bw0, bb0, bw1, bb1, bw2, bb2,
                       w0x, w0e, b0, w1, b1, w2, b2, w3, b3, w4, b4)
        return pl.pallas_call(
            _dense_body,
            grid=(1,),
            in_specs=[
                pl.BlockSpec((BH, DENSE), lambda i: (0, 0)),
                pl.BlockSpec(memory_space=pltpu.MemorySpace.HBM),
            ] + [_vmem_full(w.shape) for w in weight_args],
            out_specs=pl.BlockSpec((BH, 1), lambda i: (0, 0)),
            out_shape=jax.ShapeDtypeStruct((BH, 1), jnp.float32),
            scratch_shapes=[
                pltpu.VMEM((NBUF, CH, SLOTP, D), jnp.float32),
                pltpu.SemaphoreType.DMA((NBUF, SUB)),
            ],
            interpret=interpret,
        )(num, emb_flat, *weight_args)

    return run


def kernel(categorical_features, numerical_features, table,
           bot_W0, bot_b0, bot_W1, bot_b1, bot_W2, bot_b2,
           top_W0, top_b0, top_W1, top_b1, top_W2, top_b2,
           top_W3, top_b3, top_W4, top_b4):
    cf = categorical_features.astype(jnp.int32)
    pad_idx = (
        jnp.arange(B * (SLOTP - SLOTS), dtype=jnp.int32).reshape(B, SLOTP - SLOTS)
        % VOCAB
    )
    idx = jnp.concatenate([cf, pad_idx], axis=1)

    bf16 = jnp.bfloat16
    w0x = top_W0[:D].astype(bf16)
    w0e = (
        jnp.zeros((NF * NF, top_W0.shape[1]), jnp.float32)
        .at[_PAIR].set(top_W0[D:])
        .astype(bf16)
    )

    run = _dense_call()
    weights = (
        bot_W0, bot_b0.reshape(1, -1), bot_W1.astype(bf16), bot_b1.reshape(1, -1),
        bot_W2.astype(bf16), bot_b2.reshape(1, -1),
        w0x, w0e, top_b0.reshape(1, -1),
        top_W1.astype(bf16), top_b1.reshape(1, -1),
        top_W2.astype(bf16), top_b2.reshape(1, -1),
        top_W3.astype(bf16), top_b3.reshape(1, -1),
        top_W4.astype(bf16), top_b4.reshape(1, -1),
    )
    outs = []
    for k in range(HALVES):
        idx_k = idx[k * BH:(k + 1) * BH].reshape(1, BH * SLOTP)
        emb_k = _sc_gather(table, idx_k)
        emb_k3 = emb_k.reshape(BH, SLOTP, D)
        num_k = numerical_features[k * BH:(k + 1) * BH]
        outs.append(run(num_k, emb_k3, *weights))
    return jnp.concatenate(outs, axis=0)


# speedup vs baseline: 1.0500x; 1.0491x over previous
import jax
import jax.numpy as jnp
import numpy as np
from jax.experimental import pallas as pl
from jax.experimental.pallas import tpu as pltpu
from jax.experimental.pallas import tpu_sc as plsc

B = 4096
VOCAB = 1000000
D = 128
SLOTS = 26
SLOTP = 32
NF = SLOTS + 1
DENSE = 13
_PAIR = np.array(
    [i * NF + j for j in range(1, NF) for i in range(j)], dtype=np.int32
)

_GATHER_WIN = 256

HALVES = 2
BH = B // HALVES
CH = 256
NCH = BH // CH
SUB = 16
SUBR = CH // SUB
NBUF = 5
PREF = 4


def _sc_gather(table, idx_flat):
    n = idx_flat.shape[1]
    mesh = plsc.VectorSubcoreMesh(core_axis_name="core", subcore_axis_name="subcore")

    @pl.kernel(out_type=jax.ShapeDtypeStruct((n, D), table.dtype), mesh=mesh)
    def gather_kernel(tab_hbm, i_hbm, o_hbm):
        def body(i_vmem, o_vmem):
            pltpu.sync_copy(tab_hbm.at[i_vmem.at[0]], o_vmem)

        pltpu.emit_pipeline(
            body,
            grid=(n // _GATHER_WIN,),
            in_specs=[pl.BlockSpec((1, _GATHER_WIN), index_map=lambda i: (0, i))],
            out_specs=[pl.BlockSpec((_GATHER_WIN, D), index_map=lambda i: (i, 0))],
            core_axis_name=("core", "subcore"),
            dimension_semantics=(pltpu.PARALLEL,),
        )(i_hbm, o_hbm)

    return gather_kernel(table, idx_flat)


def _dense_body(num_ref, emb_hbm, bw0, bb0, bw1, bb1, bw2, bb2,
                w0x, w0e, b0, w1, b1, w2, b2, w3, b3, w4, b4,
                out_ref, ebuf, sems):
    f32 = jnp.float32
    bf16 = jnp.bfloat16

    def copy(c, s):
        bslot = c % NBUF
        return pltpu.make_async_copy(
            emb_hbm.at[pl.ds(c * CH + s * SUBR, SUBR)],
            ebuf.at[bslot, pl.ds(s * SUBR, SUBR)],
            sems.at[bslot, s],
        )

    for c in range(PREF):
        for s in range(SUB):
            copy(c, s).start()

    for c in range(NCH):
        if c + PREF < NCH:
            for s in range(SUB):
                copy(c + PREF, s).start()
        for s in range(SUB):
            copy(c, s).wait()

        x = num_ref[pl.ds(c * CH, CH), :]
        x = jax.nn.relu(jnp.dot(x, bw0[...], preferred_element_type=f32) + bb0[...])
        x = jax.nn.relu(jnp.dot(x.astype(bf16), bw1[...], preferred_element_type=f32) + bb1[...])
        x = jax.nn.relu(jnp.dot(x.astype(bf16), bw2[...], preferred_element_type=f32) + bb2[...])

        emb = ebuf[c % NBUF, :, :NF, :].astype(bf16)
        sl = jax.lax.broadcasted_iota(jnp.int32, (CH, NF, D), 1)
        catb = jnp.where(sl == SLOTS, x.astype(bf16)[:, None, :], emb)
        gram = jax.lax.dot_general(
            catb, catb, (((2,), (2,)), ((0,), (0,))), preferred_element_type=f32
        )
        gf = gram.astype(bf16).reshape(CH, NF * NF)

        h = jnp.dot(x.astype(bf16), w0x[...], preferred_element_type=f32)
        h = h + jnp.dot(gf, w0e[...], preferred_element_type=f32) + b0[...]
        h = jax.nn.relu(h)
        h = jax.nn.relu(jnp.dot(h.astype(bf16), w1[...], preferred_element_type=f32) + b1[...])
        h = jax.nn.relu(jnp.dot(h.astype(bf16), w2[...], preferred_element_type=f32) + b2[...])
        h = jax.nn.relu(jnp.dot(h.astype(bf16), w3[...], preferred_element_type=f32) + b3[...])
        logit = jnp.dot(h.astype(bf16), w4[...], preferred_element_type=f32) + b4[...]
        out_ref[pl.ds(c * CH, CH), :] = jax.nn.sigmoid(logit)


def _dense_call(interpret=False):
    def _vmem_full(shape):
        nd = len(shape)
        return pl.BlockSpec(shape, (lambda i: (0,) * nd))

    def run(num, emb_flat, bw0, bb0, bw1, bb1, bw2, bb2,
            w0x, w0e, b0, w1, b1, w2, b2, w3, b3, w4, b4):
        weight_args = (bw0, bb0, bw1, bb1, bw2, bb2,
                       w0x, w0e, b0, w1, b1, w2, b2, w3, b3, w4, b4)
        return pl.pallas_call(
            _dense_body,
            grid=(1,),
            in_specs=[
                pl.BlockSpec((BH, DENSE), lambda i: (0, 0)),
                pl.BlockSpec(memory_space=pltpu.MemorySpace.HBM),
            ] + [_vmem_full(w.shape) for w in weight_args],
            out_specs=pl.BlockSpec((BH, 1), lambda i: (0, 0)),
            out_shape=jax.ShapeDtypeStruct((BH, 1), jnp.float32),
            scratch_shapes=[
                pltpu.VMEM((NBUF, CH, SLOTP, D), jnp.float32),
                pltpu.SemaphoreType.DMA((NBUF, SUB)),
            ],
            interpret=interpret,
        )(num, emb_flat, *weight_args)

    return run


def kernel(categorical_features, numerical_features, table,
           bot_W0, bot_b0, bot_W1, bot_b1, bot_W2, bot_b2,
           top_W0, top_b0, top_W1, top_b1, top_W2, top_b2,
           top_W3, top_b3, top_W4, top_b4):
    cf = categorical_features.astype(jnp.int32)
    pad_idx = (
        jnp.arange(B * (SLOTP - SLOTS), dtype=jnp.int32).reshape(B, SLOTP - SLOTS)
        % VOCAB
    )
    idx = jnp.concatenate([cf, pad_idx], axis=1)

    bf16 = jnp.bfloat16
    w0x = top_W0[:D].astype(bf16)
    w0e = (
        jnp.zeros((NF * NF, top_W0.shape[1]), jnp.float32)
        .at[_PAIR].set(top_W0[D:])
        .astype(bf16)
    )

    run = _dense_call()
    weights = (
        bot_W0, bot_b0.reshape(1, -1), bot_W1.astype(bf16), bot_b1.reshape(1, -1),
        bot_W2.astype(bf16), bot_b2.reshape(1, -1),
        w0x, w0e, top_b0.reshape(1, -1),
        top_W1.astype(bf16), top_b1.reshape(1, -1),
        top_W2.astype(bf16), top_b2.reshape(1, -1),
        top_W3.astype(bf16), top_b3.reshape(1, -1),
        top_W4.astype(bf16), top_b4.reshape(1, -1),
    )
    outs = []
    for k in range(HALVES):
        idx_k = idx[k * BH:(k + 1) * BH].reshape(1, BH * SLOTP)
        emb_k = _sc_gather(table, idx_k)
        emb_k3 = emb_k.reshape(BH, SLOTP, D)
        num_k = numerical_features[k * BH:(k + 1) * BH]
        outs.append(run(num_k, emb_k3, *weights))
    return jnp.concatenate(outs, axis=0)
